# Initial kernel scaffold; baseline (speedup 1.0000x reference)
import jax
import jax.numpy as jnp
from jax.experimental import pallas as pl


def kernel(x, edge_index, W, att_src, att_dst, bias):
    raise NotImplementedError("write your pallas kernel here")


# speedup vs baseline: 19.8849x; 19.8849x over previous
import functools

import jax
import jax.numpy as jnp
from jax import lax
from jax.experimental import pallas as pl
from jax.experimental.pallas import tpu as pltpu
from jax.experimental.pallas import tpu_sc as plsc

N_NODES = 10000
D = 128
NC, NS = 2, 16
NW = NC * NS
B_EDGE = 128
BROWS = B_EDGE // 128
DEN_ROWS = N_NODES // 16
DEN_PAD = 640


def _leaky(x):
    return jnp.where(x >= 0, x, 0.2 * x)


def _b16(v):
    return jnp.full((16,), v, jnp.int32)


def _enc_body(x_ref, wt_ref, as_ref, ad_ref, h_ref, asrc_ref, adst_ref):
    h = jnp.dot(x_ref[...], wt_ref[...], preferred_element_type=jnp.float32)
    h_ref[...] = h
    asrc_ref[...] = jnp.sum(h * as_ref[...][None, :], axis=1)
    adst_ref[...] = jnp.sum(h * ad_ref[...][None, :], axis=1)


def _encode(x, wt, att_s, att_d):
    return pl.pallas_call(
        _enc_body,
        out_shape=[
            jax.ShapeDtypeStruct((N_NODES, D), jnp.float32),
            jax.ShapeDtypeStruct((N_NODES,), jnp.float32),
            jax.ShapeDtypeStruct((N_NODES,), jnp.float32),
        ],
    )(x, wt, att_s, att_d)


def _pass1_body(h_hbm, src_hbm, dst_hbm, asrc_hbm, adst_hbm,
                acc_hbm, den_hbm, p_hbm,
                asrc_v, adst_v, idx_s, idx_d, pbuf, rows, den_priv, idxred,
                shared_acc, shared_den, sem,
                *, nblocks, rows_per_w, e2):
    c = lax.axis_index("c")
    s = lax.axis_index("s")
    w = s * NC + c
    iota = jnp.arange(16, dtype=jnp.int32)
    fzero = jnp.zeros((16,), jnp.float32)

    def zden(i, _):
        den_priv[i, pl.ds(0, 16)] = fzero
        return 0
    lax.fori_loop(0, DEN_PAD, zden, 0)

    def zrow(r, _):
        for q in range(8):
            rows[r, pl.ds(q * 16, 16)] = fzero
        return 0
    lax.fori_loop(0, B_EDGE, zrow, 0)

    base_n = s * (N_NODES // NS)
    for k in range(4):
        pltpu.sync_copy(rows, shared_acc.at[pl.ds(base_n + k * 128, 128)])
    pltpu.sync_copy(rows.at[pl.ds(0, 113)],
                    shared_acc.at[pl.ds(base_n + 512, 113)])

    @pl.when(s == 0)
    def _():
        pltpu.sync_copy(den_priv, shared_den)

    for j in range(DEN_PAD // 128):
        def zidx(g, _, j=j):
            idxred[j, pl.ds(g * 16, 16)] = j * 128 + g * 16 + iota
            return 0
        lax.fori_loop(0, 8, zidx, 0)

    pltpu.sync_copy(asrc_hbm, asrc_v)
    pltpu.sync_copy(adst_hbm, adst_v)

    def mstep(i, m):
        return jnp.maximum(m, asrc_v[i, pl.ds(0, 16)])
    mv = lax.fori_loop(1, DEN_ROWS, mstep, asrc_v[0, pl.ds(0, 16)])
    for sh in (1, 2, 4, 8):
        mv = jnp.maximum(
            mv, mv.at[iota ^ sh].get(mode="promise_in_bounds"))
    amax = mv

    plsc.subcore_barrier()

    w_row0 = w * rows_per_w

    def blk(i, _):
        base_row = w_row0 + i * BROWS
        pltpu.sync_copy(src_hbm.at[pl.ds(base_row, BROWS)], idx_s)
        pltpu.sync_copy(dst_hbm.at[pl.ds(base_row, BROWS)], idx_d)
        cps = [pltpu.async_copy(h_hbm.at[idx_s.at[j]],
                                rows.at[pl.ds(j * 128, 128)], sem)
               for j in range(BROWS)]
        for j in range(BROWS):
            def grp(g, _, j=j):
                sv = idx_s[j, pl.ds(g * 16, 16)]
                dv = idx_d[j, pl.ds(g * 16, 16)]
                asv = plsc.load_gather(
                    asrc_v,
                    [lax.shift_right_logical(sv, 4), lax.bitwise_and(sv, 15)])
                adv = plsc.load_gather(
                    adst_v,
                    [lax.shift_right_logical(dv, 4), lax.bitwise_and(dv, 15)])
                e = _leaky(asv + adv)
                mh = _leaky(amax + adv)
                p = jnp.exp(e - mh)
                ge = (base_row + j) * 128 + g * 16 + iota
                p = jnp.where(ge < e2, p, 0.0)
                pbuf[j, pl.ds(g * 16, 16)] = p
                plsc.addupdate_scatter(
                    den_priv,
                    [lax.shift_right_logical(dv, 4), lax.bitwise_and(dv, 15)],
                    p)
                return 0
            lax.fori_loop(0, 8, grp, 0)
        for cp in cps:
            cp.wait()

        def scale(r, _):
            rj = lax.shift_right_logical(r, 7)
            rc = lax.bitwise_and(r, 127)
            pj = plsc.load_gather(pbuf, [_b16(rj), _b16(rc)])
            for q in range(8):
                rows[r, pl.ds(q * 16, 16)] = rows[r, pl.ds(q * 16, 16)] * pj
            return 0
        lax.fori_loop(0, B_EDGE, scale, 0)

        for j in range(BROWS):
            pltpu.sync_copy(rows.at[pl.ds(j * 128, 128)],
                            shared_acc.at[idx_d.at[j]], add=True)
        pltpu.sync_copy(pbuf, p_hbm.at[pl.ds(base_row, BROWS)])
        return 0
    lax.fori_loop(0, nblocks, blk, 0)

    plsc.subcore_barrier()
    for j in range(DEN_PAD // 128):
        pltpu.sync_copy(den_priv.at[pl.ds(j * 128, 128)],
                        shared_den.at[idxred.at[j]], add=True)
    plsc.subcore_barrier()

    base_e = s * 624
    pltpu.sync_copy(shared_acc.at[pl.ds(base_e, 624)],
                    acc_hbm.at[c, pl.ds(base_e, 624)])

    @pl.when(s == 0)
    def _():
        pltpu.sync_copy(shared_acc.at[pl.ds(9984, 16)],
                        acc_hbm.at[c, pl.ds(9984, 16)])
        pltpu.sync_copy(shared_den.at[pl.ds(0, DEN_ROWS)], den_hbm.at[c])


def _pass1(h, srcp, dstp, a_src, a_dst, nblocks, rows_per_w, e2, e2p_rows):
    mesh = plsc.VectorSubcoreMesh(core_axis_name="c", subcore_axis_name="s")
    body = functools.partial(_pass1_body, nblocks=nblocks,
                             rows_per_w=rows_per_w, e2=e2)
    return pl.kernel(
        body,
        out_type=(
            jax.ShapeDtypeStruct((NC, N_NODES, D), jnp.float32),
            jax.ShapeDtypeStruct((NC, DEN_ROWS, 16), jnp.float32),
            jax.ShapeDtypeStruct((e2p_rows, 128), jnp.float32),
        ),
        mesh=mesh,
        compiler_params=pltpu.CompilerParams(needs_layout_passes=False, use_tc_tiling_on_sc=False),
        scratch_types=[
            pltpu.VMEM((DEN_ROWS, 16), jnp.float32),
            pltpu.VMEM((DEN_ROWS, 16), jnp.float32),
            pltpu.VMEM((BROWS, 128), jnp.int32),
            pltpu.VMEM((BROWS, 128), jnp.int32),
            pltpu.VMEM((BROWS, 128), jnp.float32),
            pltpu.VMEM((B_EDGE, 128), jnp.float32),
            pltpu.VMEM((DEN_PAD, 16), jnp.float32),
            pltpu.VMEM((DEN_PAD // 128, 128), jnp.int32),
            pltpu.VMEM_SHARED((N_NODES, D), jnp.float32),
            pltpu.VMEM_SHARED((DEN_PAD, 16), jnp.float32),
            pltpu.SemaphoreType.DMA,
        ],
    )(h, srcp, dstp, a_src, a_dst)


def _pass2_body(dst_hbm, p_hbm, den_hbm, alpha_hbm,
                d0, dsum, idx_d, pbuf, abuf,
                *, nblocks, rows_per_w):
    c = lax.axis_index("c")
    s = lax.axis_index("s")
    w = s * NC + c
    pltpu.sync_copy(den_hbm.at[0], d0)
    pltpu.sync_copy(den_hbm.at[1], dsum)

    def addrow(i, _):
        dsum[i, pl.ds(0, 16)] = dsum[i, pl.ds(0, 16)] + d0[i, pl.ds(0, 16)]
        return 0
    lax.fori_loop(0, DEN_ROWS, addrow, 0)

    w_row0 = w * rows_per_w

    def blk(i, _):
        base_row = w_row0 + i * BROWS
        pltpu.sync_copy(dst_hbm.at[pl.ds(base_row, BROWS)], idx_d)
        pltpu.sync_copy(p_hbm.at[pl.ds(base_row, BROWS)], pbuf)
        for j in range(BROWS):
            def grp(g, _, j=j):
                dv = idx_d[j, pl.ds(g * 16, 16)]
                pv = pbuf[j, pl.ds(g * 16, 16)]
                den = plsc.load_gather(
                    dsum,
                    [lax.shift_right_logical(dv, 4), lax.bitwise_and(dv, 15)])
                abuf[j, pl.ds(g * 16, 16)] = pv / (den + 1e-16)
                return 0
            lax.fori_loop(0, 8, grp, 0)
        pltpu.sync_copy(abuf, alpha_hbm.at[pl.ds(base_row, BROWS)])
        return 0
    lax.fori_loop(0, nblocks, blk, 0)


def _pass2(dstp, p, den, nblocks, rows_per_w, e2p_rows):
    mesh = plsc.VectorSubcoreMesh(core_axis_name="c", subcore_axis_name="s")
    body = functools.partial(_pass2_body, nblocks=nblocks,
                             rows_per_w=rows_per_w)
    return pl.kernel(
        body,
        out_type=jax.ShapeDtypeStruct((e2p_rows, 128), jnp.float32),
        mesh=mesh,
        compiler_params=pltpu.CompilerParams(needs_layout_passes=False, use_tc_tiling_on_sc=False),
        scratch_types=[
            pltpu.VMEM((DEN_ROWS, 16), jnp.float32),
            pltpu.VMEM((DEN_ROWS, 16), jnp.float32),
            pltpu.VMEM((BROWS, 128), jnp.int32),
            pltpu.VMEM((BROWS, 128), jnp.float32),
            pltpu.VMEM((BROWS, 128), jnp.float32),
        ],
    )(dstp, p, den)


def _fin_body(acc_ref, den_ref, bias_ref, out_ref):
    a = acc_ref[0] + acc_ref[1]
    d = den_ref[0] + den_ref[1]
    r = 1.0 / (d + 1e-16)
    o = a * r[:, None] + bias_ref[...][None, :]
    out_ref[...] = jnp.where(o > 0, o, jnp.exp(jnp.minimum(o, 0.0)) - 1.0)


def _final(acc, den2, bias):
    return pl.pallas_call(
        _fin_body,
        out_shape=jax.ShapeDtypeStruct((N_NODES, D), jnp.float32),
    )(acc, den2, bias)


def kernel(x, edge_index, W, att_src, att_dst, bias):
    loop = jnp.arange(N_NODES, dtype=edge_index.dtype)
    src2 = jnp.concatenate([edge_index[0], loop])
    dst2 = jnp.concatenate([edge_index[1], loop])
    e2 = src2.shape[0]

    nblocks = pl.cdiv(e2, NW * B_EDGE)
    e2p = NW * B_EDGE * nblocks
    rows_per_w = (e2p // 128) // NW
    pad = e2p - e2
    srcp = jnp.concatenate([src2, jnp.zeros((pad,), src2.dtype)])
    dstp = jnp.concatenate([dst2, jnp.zeros((pad,), dst2.dtype)])
    srcp = srcp.reshape(e2p // 128, 128)
    dstp = dstp.reshape(e2p // 128, 128)

    h, a_src, a_dst = _encode(x, W.T, att_src, att_dst)
    acc, den, p = _pass1(h, srcp, dstp,
                         a_src.reshape(DEN_ROWS, 16), a_dst.reshape(DEN_ROWS, 16),
                         nblocks, rows_per_w, e2, e2p // 128)
    alpha_p = _pass2(dstp, p, den, nblocks, rows_per_w, e2p // 128)
    alpha = alpha_p.reshape(e2p)[:e2]
    out = _final(acc, den.reshape(NC, N_NODES), bias)
    new_edge_index = jnp.stack([src2, dst2])
    return out, new_edge_index, alpha


# speedup vs baseline: 22.7328x; 1.1432x over previous
import functools

import jax
import jax.numpy as jnp
from jax import lax
from jax.experimental import pallas as pl
from jax.experimental.pallas import tpu as pltpu
from jax.experimental.pallas import tpu_sc as plsc

N_NODES = 10000
D = 128
NC, NS = 2, 16
NW = NC * NS
B_EDGE = 128
BROWS = B_EDGE // 128
DEN_ROWS = N_NODES // 16
DEN_PAD = 640


def _leaky(x):
    return jnp.where(x >= 0, x, 0.2 * x)


def _b16(v):
    return jnp.full((16,), v, jnp.int32)


def _enc_body(x_ref, wt_ref, as_ref, ad_ref, h_ref, asrc_ref, adst_ref):
    h = jnp.dot(x_ref[...], wt_ref[...], preferred_element_type=jnp.float32)
    h_ref[...] = h
    asrc_ref[...] = jnp.sum(h * as_ref[...][None, :], axis=1)
    adst_ref[...] = jnp.sum(h * ad_ref[...][None, :], axis=1)


def _encode(x, wt, att_s, att_d):
    return pl.pallas_call(
        _enc_body,
        out_shape=[
            jax.ShapeDtypeStruct((N_NODES, D), jnp.float32),
            jax.ShapeDtypeStruct((N_NODES,), jnp.float32),
            jax.ShapeDtypeStruct((N_NODES,), jnp.float32),
        ],
    )(x, wt, att_s, att_d)


def _pass1_body(h_hbm, src_hbm, dst_hbm, asrc_hbm, adst_hbm,
                acc_hbm, den_hbm, p_hbm,
                asrc_v, adst_v, idx_s, idx_d, pbuf, rows, den_priv, idxred,
                shared_acc, shared_den, sem,
                *, nblocks, rows_per_w, e2):
    c = lax.axis_index("c")
    s = lax.axis_index("s")
    w = s * NC + c
    iota = jnp.arange(16, dtype=jnp.int32)
    fzero = jnp.zeros((16,), jnp.float32)

    def zden(i, _):
        den_priv[i, pl.ds(0, 16)] = fzero
        return 0
    lax.fori_loop(0, DEN_PAD, zden, 0)

    def zrow(r, _):
        for q in range(8):
            rows[r, pl.ds(q * 16, 16)] = fzero
        return 0
    lax.fori_loop(0, B_EDGE, zrow, 0)

    base_n = s * (N_NODES // NS)
    for k in range(4):
        pltpu.sync_copy(rows, shared_acc.at[pl.ds(base_n + k * 128, 128)])
    pltpu.sync_copy(rows.at[pl.ds(0, 113)],
                    shared_acc.at[pl.ds(base_n + 512, 113)])

    @pl.when(s == 0)
    def _():
        pltpu.sync_copy(den_priv, shared_den)

    for j in range(DEN_PAD // 128):
        def zidx(g, _, j=j):
            idxred[j, pl.ds(g * 16, 16)] = j * 128 + g * 16 + iota
            return 0
        lax.fori_loop(0, 8, zidx, 0)

    pltpu.sync_copy(asrc_hbm, asrc_v)
    pltpu.sync_copy(adst_hbm, adst_v)

    def mstep(i, m):
        return jnp.maximum(m, asrc_v[i, pl.ds(0, 16)])
    mv = lax.fori_loop(1, DEN_ROWS, mstep, asrc_v[0, pl.ds(0, 16)])
    for sh in (1, 2, 4, 8):
        mv = jnp.maximum(
            mv, mv.at[iota ^ sh].get(mode="promise_in_bounds"))
    amax = mv

    plsc.subcore_barrier()

    w_row0 = w * rows_per_w

    def blk(i, _):
        base_row = w_row0 + i * BROWS
        pltpu.sync_copy(src_hbm.at[pl.ds(base_row, BROWS)], idx_s)
        pltpu.sync_copy(dst_hbm.at[pl.ds(base_row, BROWS)], idx_d)
        cps = [pltpu.async_copy(h_hbm.at[idx_s.at[j]],
                                rows.at[pl.ds(j * 128, 128)], sem)
               for j in range(BROWS)]
        for j in range(BROWS):
            def grp(g, _, j=j):
                sv = idx_s[j, pl.ds(g * 16, 16)]
                dv = idx_d[j, pl.ds(g * 16, 16)]
                asv = plsc.load_gather(
                    asrc_v,
                    [lax.shift_right_logical(sv, 4), lax.bitwise_and(sv, 15)])
                adv = plsc.load_gather(
                    adst_v,
                    [lax.shift_right_logical(dv, 4), lax.bitwise_and(dv, 15)])
                e = _leaky(asv + adv)
                mh = _leaky(amax + adv)
                p = jnp.exp(e - mh)
                ge = (base_row + j) * 128 + g * 16 + iota
                p = jnp.where(ge < e2, p, 0.0)
                pbuf[j, pl.ds(g * 16, 16)] = p
                plsc.addupdate_scatter(
                    den_priv,
                    [lax.shift_right_logical(dv, 4), lax.bitwise_and(dv, 15)],
                    p)
                return 0
            lax.fori_loop(0, 8, grp, 0)
        for cp in cps:
            cp.wait()

        def scale(r, _):
            rj = lax.shift_right_logical(r, 7)
            rc = lax.bitwise_and(r, 127)
            pj = plsc.load_gather(pbuf, [_b16(rj), _b16(rc)])
            for q in range(8):
                rows[r, pl.ds(q * 16, 16)] = rows[r, pl.ds(q * 16, 16)] * pj
            return 0
        lax.fori_loop(0, B_EDGE, scale, 0)

        for j in range(BROWS):
            pltpu.sync_copy(rows.at[pl.ds(j * 128, 128)],
                            shared_acc.at[idx_d.at[j]], add=True)
        pltpu.sync_copy(pbuf, p_hbm.at[pl.ds(base_row, BROWS)])
        return 0
    lax.fori_loop(0, nblocks, blk, 0)

    plsc.subcore_barrier()
    for j in range(DEN_PAD // 128):
        pltpu.sync_copy(den_priv.at[pl.ds(j * 128, 128)],
                        shared_den.at[idxred.at[j]], add=True)
    plsc.subcore_barrier()

    base_e = s * 624
    pltpu.sync_copy(shared_acc.at[pl.ds(base_e, 624)],
                    acc_hbm.at[c, pl.ds(base_e, 624)])

    @pl.when(s == 0)
    def _():
        pltpu.sync_copy(shared_acc.at[pl.ds(9984, 16)],
                        acc_hbm.at[c, pl.ds(9984, 16)])
        pltpu.sync_copy(shared_den.at[pl.ds(0, DEN_ROWS)], den_hbm.at[c])


def _pass1(h, srcp, dstp, a_src, a_dst, nblocks, rows_per_w, e2, e2p_rows):
    mesh = plsc.VectorSubcoreMesh(core_axis_name="c", subcore_axis_name="s")
    body = functools.partial(_pass1_body, nblocks=nblocks,
                             rows_per_w=rows_per_w, e2=e2)
    return pl.kernel(
        body,
        out_type=(
            jax.ShapeDtypeStruct((NC, N_NODES, D), jnp.float32),
            jax.ShapeDtypeStruct((NC, DEN_ROWS, 16), jnp.float32),
            jax.ShapeDtypeStruct((e2p_rows, 128), jnp.float32),
        ),
        mesh=mesh,
        compiler_params=pltpu.CompilerParams(needs_layout_passes=False, use_tc_tiling_on_sc=False),
        scratch_types=[
            pltpu.VMEM((DEN_ROWS, 16), jnp.float32),
            pltpu.VMEM((DEN_ROWS, 16), jnp.float32),
            pltpu.VMEM((BROWS, 128), jnp.int32),
            pltpu.VMEM((BROWS, 128), jnp.int32),
            pltpu.VMEM((BROWS, 128), jnp.float32),
            pltpu.VMEM((B_EDGE, 128), jnp.float32),
            pltpu.VMEM((DEN_PAD, 16), jnp.float32),
            pltpu.VMEM((DEN_PAD // 128, 128), jnp.int32),
            pltpu.VMEM_SHARED((N_NODES, D), jnp.float32),
            pltpu.VMEM_SHARED((DEN_PAD, 16), jnp.float32),
            pltpu.SemaphoreType.DMA,
        ],
    )(h, srcp, dstp, a_src, a_dst)


P2R = 41


def _pass2_body(dst_hbm, p_hbm, den_hbm, alpha_hbm,
                d0, dsum, idx_d, pbuf, abuf,
                *, nblocks, rows_per_w):
    c = lax.axis_index("c")
    s = lax.axis_index("s")
    w = s * NC + c
    pltpu.sync_copy(den_hbm.at[0], d0)
    pltpu.sync_copy(den_hbm.at[1], dsum)

    def addrow(i, _):
        dsum[i, pl.ds(0, 16)] = dsum[i, pl.ds(0, 16)] + d0[i, pl.ds(0, 16)]
        return 0
    lax.fori_loop(0, DEN_ROWS, addrow, 0)

    w_row0 = w * rows_per_w

    for i in range(rows_per_w // P2R):
        base_row = w_row0 + i * P2R
        pltpu.sync_copy(dst_hbm.at[pl.ds(base_row, P2R)], idx_d)
        pltpu.sync_copy(p_hbm.at[pl.ds(base_row, P2R)], pbuf)

        @plsc.parallel_loop(0, P2R * 8, unroll=4)
        def _grp(g):
            j = lax.shift_right_logical(g, 3)
            q = lax.bitwise_and(g, 7)
            dv = idx_d[j, pl.ds(q * 16, 16)]
            pv = pbuf[j, pl.ds(q * 16, 16)]
            den = plsc.load_gather(
                dsum,
                [lax.shift_right_logical(dv, 4), lax.bitwise_and(dv, 15)])
            abuf[j, pl.ds(q * 16, 16)] = pv / (den + 1e-16)
        pltpu.sync_copy(abuf, alpha_hbm.at[pl.ds(base_row, P2R)])


def _pass2(dstp, p, den, nblocks, rows_per_w, e2p_rows):
    mesh = plsc.VectorSubcoreMesh(core_axis_name="c", subcore_axis_name="s")
    body = functools.partial(_pass2_body, nblocks=nblocks,
                             rows_per_w=rows_per_w)
    return pl.kernel(
        body,
        out_type=jax.ShapeDtypeStruct((e2p_rows, 128), jnp.float32),
        mesh=mesh,
        compiler_params=pltpu.CompilerParams(needs_layout_passes=False, use_tc_tiling_on_sc=False),
        scratch_types=[
            pltpu.VMEM((DEN_ROWS, 16), jnp.float32),
            pltpu.VMEM((DEN_ROWS, 16), jnp.float32),
            pltpu.VMEM((P2R, 128), jnp.int32),
            pltpu.VMEM((P2R, 128), jnp.float32),
            pltpu.VMEM((P2R, 128), jnp.float32),
        ],
    )(dstp, p, den)


def _fin_body(acc_ref, den_ref, bias_ref, out_ref):
    a = acc_ref[0] + acc_ref[1]
    d = den_ref[0] + den_ref[1]
    r = 1.0 / (d + 1e-16)
    o = a * r[:, None] + bias_ref[...][None, :]
    out_ref[...] = jnp.where(o > 0, o, jnp.exp(jnp.minimum(o, 0.0)) - 1.0)


def _final(acc, den2, bias):
    return pl.pallas_call(
        _fin_body,
        out_shape=jax.ShapeDtypeStruct((N_NODES, D), jnp.float32),
    )(acc, den2, bias)


def kernel(x, edge_index, W, att_src, att_dst, bias):
    loop = jnp.arange(N_NODES, dtype=edge_index.dtype)
    src2 = jnp.concatenate([edge_index[0], loop])
    dst2 = jnp.concatenate([edge_index[1], loop])
    e2 = src2.shape[0]

    nblocks = pl.cdiv(e2, NW * B_EDGE)
    e2p = NW * B_EDGE * nblocks
    rows_per_w = (e2p // 128) // NW
    pad = e2p - e2
    srcp = jnp.concatenate([src2, jnp.zeros((pad,), src2.dtype)])
    dstp = jnp.concatenate([dst2, jnp.zeros((pad,), dst2.dtype)])
    srcp = srcp.reshape(e2p // 128, 128)
    dstp = dstp.reshape(e2p // 128, 128)

    h, a_src, a_dst = _encode(x, W.T, att_src, att_dst)
    acc, den, p = _pass1(h, srcp, dstp,
                         a_src.reshape(DEN_ROWS, 16), a_dst.reshape(DEN_ROWS, 16),
                         nblocks, rows_per_w, e2, e2p // 128)
    alpha_p = _pass2(dstp, p, den, nblocks, rows_per_w, e2p // 128)
    alpha = alpha_p.reshape(e2p)[:e2]
    out = _final(acc, den.reshape(NC, N_NODES), bias)
    new_edge_index = jnp.stack([src2, dst2])
    return out, new_edge_index, alpha


# speedup vs baseline: 25.8000x; 1.1349x over previous
import functools

import jax
import jax.numpy as jnp
from jax import lax
from jax.experimental import pallas as pl
from jax.experimental.pallas import tpu as pltpu
from jax.experimental.pallas import tpu_sc as plsc

N_NODES = 10000
D = 128
NC, NS = 2, 16
NW = NC * NS
B_EDGE = 128
BROWS = B_EDGE // 128
DEN_ROWS = N_NODES // 16
DEN_PAD = 640


def _leaky(x):
    return jnp.where(x >= 0, x, 0.2 * x)


def _b16(v):
    return jnp.full((16,), v, jnp.int32)


def _enc_body(x_ref, wt_ref, as_ref, ad_ref, h_ref, asrc_ref, adst_ref):
    h = jnp.dot(x_ref[...], wt_ref[...], preferred_element_type=jnp.float32)
    h_ref[...] = h
    asrc_ref[...] = jnp.sum(h * as_ref[...][None, :], axis=1)
    adst_ref[...] = jnp.sum(h * ad_ref[...][None, :], axis=1)


def _encode(x, wt, att_s, att_d):
    return pl.pallas_call(
        _enc_body,
        out_shape=[
            jax.ShapeDtypeStruct((N_NODES, D), jnp.float32),
            jax.ShapeDtypeStruct((N_NODES,), jnp.float32),
            jax.ShapeDtypeStruct((N_NODES,), jnp.float32),
        ],
    )(x, wt, att_s, att_d)


def _pass1_body(h_hbm, src_hbm, dst_hbm, asrc_hbm, adst_hbm,
                acc_hbm, den_hbm, p_hbm,
                asrc_v, adst_v, idx_s, idx_d, pbuf, rows, den_priv, idxred,
                shared_acc, shared_den, sem, sem_s,
                *, nblocks, rows_per_w, e2):
    c = lax.axis_index("c")
    s = lax.axis_index("s")
    w = s * NC + c
    iota = jnp.arange(16, dtype=jnp.int32)
    fzero = jnp.zeros((16,), jnp.float32)

    def zden(i, _):
        den_priv[i, pl.ds(0, 16)] = fzero
        return 0
    lax.fori_loop(0, DEN_PAD, zden, 0)

    def zrow(r, _):
        for q in range(8):
            rows[r, pl.ds(q * 16, 16)] = fzero
        return 0
    lax.fori_loop(0, B_EDGE, zrow, 0)

    base_n = s * (N_NODES // NS)
    for k in range(4):
        pltpu.sync_copy(rows, shared_acc.at[pl.ds(base_n + k * 128, 128)])
    pltpu.sync_copy(rows.at[pl.ds(0, 113)],
                    shared_acc.at[pl.ds(base_n + 512, 113)])

    @pl.when(s == 0)
    def _():
        pltpu.sync_copy(den_priv, shared_den)

    for j in range(DEN_PAD // 128):
        def zidx(g, _, j=j):
            idxred[j, pl.ds(g * 16, 16)] = j * 128 + g * 16 + iota
            return 0
        lax.fori_loop(0, 8, zidx, 0)

    pltpu.sync_copy(asrc_hbm, asrc_v)
    pltpu.sync_copy(adst_hbm, adst_v)

    def mstep(i, m):
        return jnp.maximum(m, asrc_v[i, pl.ds(0, 16)])
    mv = lax.fori_loop(1, DEN_ROWS, mstep, asrc_v[0, pl.ds(0, 16)])
    for sh in (1, 2, 4, 8):
        mv = jnp.maximum(
            mv, mv.at[iota ^ sh].get(mode="promise_in_bounds"))
    amax = mv

    plsc.subcore_barrier()

    w_row0 = w * rows_per_w

    def blk(i, _):
        par = lax.bitwise_and(i, 1)
        base_row = w_row0 + i
        pltpu.sync_copy(src_hbm.at[pl.ds(base_row, 1)], idx_s)
        pltpu.sync_copy(dst_hbm.at[pl.ds(base_row, 1)],
                        idx_d.at[pl.ds(par, 1)])

        @pl.when(i > 0)
        def _():
            pltpu.make_async_copy(
                rows, shared_acc.at[pl.ds(0, B_EDGE)], sem_s).wait()
        cp = pltpu.async_copy(h_hbm.at[idx_s.at[0]], rows, sem)

        def grp(g, _):
            sv = idx_s[0, pl.ds(g * 16, 16)]
            dv = idx_d[par, pl.ds(g * 16, 16)]
            asv = plsc.load_gather(
                asrc_v,
                [lax.shift_right_logical(sv, 4), lax.bitwise_and(sv, 15)])
            adv = plsc.load_gather(
                adst_v,
                [lax.shift_right_logical(dv, 4), lax.bitwise_and(dv, 15)])
            e = _leaky(asv + adv)
            mh = _leaky(amax + adv)
            p = jnp.exp(e - mh)
            ge = base_row * 128 + g * 16 + iota
            p = jnp.where(ge < e2, p, 0.0)
            pbuf[0, pl.ds(g * 16, 16)] = p
            plsc.addupdate_scatter(
                den_priv,
                [lax.shift_right_logical(dv, 4), lax.bitwise_and(dv, 15)],
                p)
            return 0
        lax.fori_loop(0, 8, grp, 0)
        cp.wait()

        def scale(t, _):
            for u in range(4):
                r = t * 4 + u
                pj = plsc.load_gather(pbuf, [_b16(0), _b16(r)])
                for q in range(8):
                    rows[r, pl.ds(q * 16, 16)] = (
                        rows[r, pl.ds(q * 16, 16)] * pj)
            return 0
        lax.fori_loop(0, B_EDGE // 4, scale, 0)

        pltpu.async_copy(rows, shared_acc.at[idx_d.at[par]], sem_s, add=True)
        pltpu.sync_copy(pbuf, p_hbm.at[pl.ds(base_row, 1)])
        return 0
    lax.fori_loop(0, nblocks, blk, 0)
    pltpu.make_async_copy(rows, shared_acc.at[pl.ds(0, B_EDGE)], sem_s).wait()

    plsc.subcore_barrier()
    for j in range(DEN_PAD // 128):
        pltpu.sync_copy(den_priv.at[pl.ds(j * 128, 128)],
                        shared_den.at[idxred.at[j]], add=True)
    plsc.subcore_barrier()

    base_e = s * 624
    pltpu.sync_copy(shared_acc.at[pl.ds(base_e, 624)],
                    acc_hbm.at[c, pl.ds(base_e, 624)])

    @pl.when(s == 0)
    def _():
        pltpu.sync_copy(shared_acc.at[pl.ds(9984, 16)],
                        acc_hbm.at[c, pl.ds(9984, 16)])
        pltpu.sync_copy(shared_den.at[pl.ds(0, DEN_ROWS)], den_hbm.at[c])


def _pass1(h, srcp, dstp, a_src, a_dst, nblocks, rows_per_w, e2, e2p_rows):
    mesh = plsc.VectorSubcoreMesh(core_axis_name="c", subcore_axis_name="s")
    body = functools.partial(_pass1_body, nblocks=nblocks,
                             rows_per_w=rows_per_w, e2=e2)
    return pl.kernel(
        body,
        out_type=(
            jax.ShapeDtypeStruct((NC, N_NODES, D), jnp.float32),
            jax.ShapeDtypeStruct((NC, DEN_ROWS, 16), jnp.float32),
            jax.ShapeDtypeStruct((e2p_rows, 128), jnp.float32),
        ),
        mesh=mesh,
        compiler_params=pltpu.CompilerParams(needs_layout_passes=False, use_tc_tiling_on_sc=False),
        scratch_types=[
            pltpu.VMEM((DEN_ROWS, 16), jnp.float32),
            pltpu.VMEM((DEN_ROWS, 16), jnp.float32),
            pltpu.VMEM((BROWS, 128), jnp.int32),
            pltpu.VMEM((2, 128), jnp.int32),
            pltpu.VMEM((BROWS, 128), jnp.float32),
            pltpu.VMEM((B_EDGE, 128), jnp.float32),
            pltpu.VMEM((DEN_PAD, 16), jnp.float32),
            pltpu.VMEM((DEN_PAD // 128, 128), jnp.int32),
            pltpu.VMEM_SHARED((N_NODES, D), jnp.float32),
            pltpu.VMEM_SHARED((DEN_PAD, 16), jnp.float32),
            pltpu.SemaphoreType.DMA,
            pltpu.SemaphoreType.DMA,
        ],
    )(h, srcp, dstp, a_src, a_dst)


P2R = 41


def _pass2_body(dst_hbm, p_hbm, den_hbm, alpha_hbm,
                d0, dsum, idx_d, pbuf, abuf,
                *, nblocks, rows_per_w):
    c = lax.axis_index("c")
    s = lax.axis_index("s")
    w = s * NC + c
    pltpu.sync_copy(den_hbm.at[0], d0)
    pltpu.sync_copy(den_hbm.at[1], dsum)

    def addrow(i, _):
        dsum[i, pl.ds(0, 16)] = dsum[i, pl.ds(0, 16)] + d0[i, pl.ds(0, 16)]
        return 0
    lax.fori_loop(0, DEN_ROWS, addrow, 0)

    w_row0 = w * rows_per_w

    for i in range(rows_per_w // P2R):
        base_row = w_row0 + i * P2R
        pltpu.sync_copy(dst_hbm.at[pl.ds(base_row, P2R)], idx_d)
        pltpu.sync_copy(p_hbm.at[pl.ds(base_row, P2R)], pbuf)

        @plsc.parallel_loop(0, P2R * 8, unroll=4)
        def _grp(g):
            j = lax.shift_right_logical(g, 3)
            q = lax.bitwise_and(g, 7)
            dv = idx_d[j, pl.ds(q * 16, 16)]
            pv = pbuf[j, pl.ds(q * 16, 16)]
            den = plsc.load_gather(
                dsum,
                [lax.shift_right_logical(dv, 4), lax.bitwise_and(dv, 15)])
            abuf[j, pl.ds(q * 16, 16)] = pv / (den + 1e-16)
        pltpu.sync_copy(abuf, alpha_hbm.at[pl.ds(base_row, P2R)])


def _pass2(dstp, p, den, nblocks, rows_per_w, e2p_rows):
    mesh = plsc.VectorSubcoreMesh(core_axis_name="c", subcore_axis_name="s")
    body = functools.partial(_pass2_body, nblocks=nblocks,
                             rows_per_w=rows_per_w)
    return pl.kernel(
        body,
        out_type=jax.ShapeDtypeStruct((e2p_rows, 128), jnp.float32),
        mesh=mesh,
        compiler_params=pltpu.CompilerParams(needs_layout_passes=False, use_tc_tiling_on_sc=False),
        scratch_types=[
            pltpu.VMEM((DEN_ROWS, 16), jnp.float32),
            pltpu.VMEM((DEN_ROWS, 16), jnp.float32),
            pltpu.VMEM((P2R, 128), jnp.int32),
            pltpu.VMEM((P2R, 128), jnp.float32),
            pltpu.VMEM((P2R, 128), jnp.float32),
        ],
    )(dstp, p, den)


def _fin_body(acc_ref, den_ref, bias_ref, out_ref):
    a = acc_ref[0] + acc_ref[1]
    d = den_ref[0] + den_ref[1]
    r = 1.0 / (d + 1e-16)
    o = a * r[:, None] + bias_ref[...][None, :]
    out_ref[...] = jnp.where(o > 0, o, jnp.exp(jnp.minimum(o, 0.0)) - 1.0)


def _final(acc, den2, bias):
    return pl.pallas_call(
        _fin_body,
        out_shape=jax.ShapeDtypeStruct((N_NODES, D), jnp.float32),
    )(acc, den2, bias)


def kernel(x, edge_index, W, att_src, att_dst, bias):
    loop = jnp.arange(N_NODES, dtype=edge_index.dtype)
    src2 = jnp.concatenate([edge_index[0], loop])
    dst2 = jnp.concatenate([edge_index[1], loop])
    e2 = src2.shape[0]

    nblocks = pl.cdiv(e2, NW * B_EDGE)
    e2p = NW * B_EDGE * nblocks
    rows_per_w = (e2p // 128) // NW
    pad = e2p - e2
    srcp = jnp.concatenate([src2, jnp.zeros((pad,), src2.dtype)])
    dstp = jnp.concatenate([dst2, jnp.zeros((pad,), dst2.dtype)])
    srcp = srcp.reshape(e2p // 128, 128)
    dstp = dstp.reshape(e2p // 128, 128)

    h, a_src, a_dst = _encode(x, W.T, att_src, att_dst)
    acc, den, p = _pass1(h, srcp, dstp,
                         a_src.reshape(DEN_ROWS, 16), a_dst.reshape(DEN_ROWS, 16),
                         nblocks, rows_per_w, e2, e2p // 128)
    alpha_p = _pass2(dstp, p, den, nblocks, rows_per_w, e2p // 128)
    alpha = alpha_p.reshape(e2p)[:e2]
    out = _final(acc, den.reshape(NC, N_NODES), bias)
    new_edge_index = jnp.stack([src2, dst2])
    return out, new_edge_index, alpha
